# Initial kernel scaffold; baseline (speedup 1.0000x reference)
import jax
import jax.numpy as jnp
from jax.experimental import pallas as pl


def kernel(init_node_emb, init_edge_emb, head_ind, tail_ind, is_training, gt_node_dists, gt_edge_dists, Wq, Wk, Wv, We, Wo, Weo, W_ecls, b_ecls, W_ncls, b_ncls, edge_table, node_table):
    raise NotImplementedError("write your pallas kernel here")


# speedup vs baseline: 10.6085x; 10.6085x over previous
import functools

import jax
import jax.numpy as jnp
from jax import lax
from jax.experimental import pallas as pl
from jax.experimental.pallas import tpu as pltpu
from jax.experimental.pallas import tpu_sc as plsc

_D = 128
_H = 8
_DH = 16
_N = 10000
_E = 160000
_NEC = 51
_NNC = 151
_L = 2

_NC = 2
_NS = 16
_NW = _NC * _NS
_CH = 128
_NCHT = _E // _CH

_GW = 16
_NGRP = _D // _GW + 1
_NHALF = 2
_HN = 5120
_TR = _HN
_HNP = _HN + 8
_SCH = 1600
_NSCH = _E // _SCH

_BN = 1000
_BE = _SCH
_BO = 320


def _head_sum_mat():
    d = lax.broadcasted_iota(jnp.int32, (_D, _H), 0)
    h = lax.broadcasted_iota(jnp.int32, (_D, _H), 1)
    return (d // _DH == h).astype(jnp.float32)


def _head_bcast_mat():
    h = lax.broadcasted_iota(jnp.int32, (_H, _D), 0)
    d = lax.broadcasted_iota(jnp.int32, (_H, _D), 1)
    return (d // _DH == h).astype(jnp.float32)


def _qkv_body(x_ref, wq_ref, wk_ref, wv_ref, q_ref, k_ref, v_ref):
    x = x_ref[...]
    q_ref[...] = jnp.dot(x, wq_ref[...], preferred_element_type=jnp.float32)
    k_ref[...] = jnp.dot(x, wk_ref[...], preferred_element_type=jnp.float32)
    v_ref[...] = jnp.dot(x, wv_ref[...], preferred_element_type=jnp.float32)


def _tc_qkv(x, wq, wk, wv):
    n = x.shape[0]
    bs_x = pl.BlockSpec((_BN, _D), lambda i: (i, 0))
    bs_w = pl.BlockSpec((_D, _D), lambda i: (0, 0))
    return pl.pallas_call(
        _qkv_body,
        grid=(n // _BN,),
        in_specs=[bs_x, bs_w, bs_w, bs_w],
        out_specs=[bs_x, bs_x, bs_x],
        out_shape=[jax.ShapeDtypeStruct((n, _D), jnp.float32)] * 3,
    )(x, wq, wk, wv)


def _edge_body(e_ref, q_ref, k_ref, v_ref, we_ref, weo_ref, ne_ref, *g_refs):
    edge = e_ref[...]
    q = q_ref[...]
    k = k_ref[...]
    v = v_ref[...]
    ee = jnp.dot(edge, we_ref[...], preferred_element_type=jnp.float32)
    kee = k + ee
    score = jnp.dot(q * kee, _head_sum_mat(),
                    preferred_element_type=jnp.float32) * 0.25
    ex = jnp.exp(score)
    exb = jnp.dot(ex, _head_bcast_mat(), preferred_element_type=jnp.float32)
    num = (v + ee) * exb
    num_ref, exp_ref = g_refs
    num_ref[...] = num
    exp_ref[...] = jnp.dot(ex, _head_exp_mat(),
                           preferred_element_type=jnp.float32)
    s = q + kee
    eo = jnp.dot(s, weo_ref[...], preferred_element_type=jnp.float32)
    ne_ref[...] = jnp.where(eo >= 0, eo, 0.2 * eo) + edge


def _head_exp_mat():
    h = lax.broadcasted_iota(jnp.int32, (_H, _D), 0)
    d = lax.broadcasted_iota(jnp.int32, (_H, _D), 1)
    return (d == h).astype(jnp.float32)


def _tc_edge(ze, q, k, v, we, weo):
    bs_e = pl.BlockSpec((_BE, _D), lambda i: (i, 0))
    bs_w = pl.BlockSpec((_D, _D), lambda i: (0, 0))
    sd_e = jax.ShapeDtypeStruct((_E, _D), jnp.float32)
    return pl.pallas_call(
        _edge_body,
        grid=(_E // _BE,),
        in_specs=[bs_e, bs_e, bs_e, bs_e, bs_w, bs_w],
        out_specs=[bs_e, bs_e, bs_e],
        out_shape=[sd_e, sd_e, sd_e],
    )(ze, q, k, v, we, weo)


def _psum_body(p_ref, o_ref):
    j = pl.program_id(1)
    blk = jnp.sum(p_ref[...], axis=(0, 1))

    @pl.when(j == 0)
    def _():
        o_ref[0] = blk

    @pl.when(j > 0)
    def _():
        o_ref[0] = o_ref[0] + blk


def _tc_psum(parts):
    rows = _HNP * _GW // 128
    return pl.pallas_call(
        _psum_body,
        grid=(_NHALF * _NGRP, _NW // 8),
        in_specs=[pl.BlockSpec((1, 8, rows, 128),
                               lambda i, j: (i, j, 0, 0))],
        out_specs=pl.BlockSpec((1, rows, 128), lambda i, j: (i, 0, 0)),
        out_shape=jax.ShapeDtypeStruct(
            (_NHALF * _NGRP, rows, 128), jnp.float32),
    )(parts)


def _nodeout_body(x_ref, wo_ref, *rest):
    g_refs, out_ref = rest[:_NGRP], rest[_NGRP]
    parts = [r[0] for r in g_refs]
    agg = jnp.concatenate(parts[:_NGRP - 1], axis=-1)
    den8 = parts[_NGRP - 1][:, :_H]
    denb = jnp.dot(den8, _head_bcast_mat(), preferred_element_type=jnp.float32)
    aggn = agg / (denb + 1e-9)
    h = jnp.dot(aggn, wo_ref[...], preferred_element_type=jnp.float32)
    out_ref[...] = jnp.where(h >= 0, h, 0.2 * h) + x_ref[...]


def _tc_nodeout(psum, node, wo):
    def spec(g):
        return pl.BlockSpec(
            (1, _BO, _GW),
            lambda i, g=g: ((i // 16) * _NGRP + g, i % 16, 0))

    return pl.pallas_call(
        _nodeout_body,
        grid=(pl.cdiv(_N, _BO),),
        in_specs=[
            pl.BlockSpec((_BO, _D), lambda i: (i, 0)),
            pl.BlockSpec((_D, _D), lambda i: (0, 0)),
        ] + [spec(g) for g in range(_NGRP)],
        out_specs=pl.BlockSpec((_BO, _D), lambda i: (i, 0)),
        out_shape=jax.ShapeDtypeStruct((_N, _D), jnp.float32),
    )(node, wo, *([psum] * _NGRP))


def _match_gt_body(t_ref, z_ref, w_ref, b_ref, gt_ref, tab_ref, a_ref, d_ref):
    z = z_ref[...]
    alpha = jnp.dot(z, w_ref[...], preferred_element_type=jnp.float32)
    alpha = alpha + b_ref[0:1, :]
    m = jnp.max(alpha, axis=-1, keepdims=True)
    e = jnp.exp(alpha - m)
    p_sm = e / jnp.sum(e, axis=-1, keepdims=True)
    g = gt_ref[...]
    p_gt = g / (jnp.sum(g, axis=-1, keepdims=True) + 1e-9)
    p = jnp.where(t_ref[0, 0] != 0, p_gt, p_sm)
    a_ref[...] = alpha
    d_ref[...] = jnp.dot(p, tab_ref[...], preferred_element_type=jnp.float32)


def _match_sm_body(z_ref, w_ref, b_ref, tab_ref, a_ref, d_ref):
    z = z_ref[...]
    alpha = jnp.dot(z, w_ref[...], preferred_element_type=jnp.float32)
    alpha = alpha + b_ref[0:1, :]
    m = jnp.max(alpha, axis=-1, keepdims=True)
    e = jnp.exp(alpha - m)
    p = e / jnp.sum(e, axis=-1, keepdims=True)
    a_ref[...] = alpha
    d_ref[...] = jnp.dot(p, tab_ref[...], preferred_element_type=jnp.float32)


def _tc_match(z, w, b8, gt, tab, t, use_gt):
    rows, c = z.shape[0], w.shape[1]
    bs_z = pl.BlockSpec((_BN, _D), lambda i: (i, 0))
    bs_w = pl.BlockSpec((_D, c), lambda i: (0, 0))
    bs_b = pl.BlockSpec((8, c), lambda i: (0, 0))
    bs_g = pl.BlockSpec((_BN, c), lambda i: (i, 0))
    bs_tab = pl.BlockSpec((c, _D), lambda i: (0, 0))
    out_shape = [
        jax.ShapeDtypeStruct((rows, c), jnp.float32),
        jax.ShapeDtypeStruct((rows, _D), jnp.float32),
    ]
    out_specs = [bs_g, bs_z]
    if use_gt:
        bs_t = pl.BlockSpec(memory_space=pltpu.SMEM)
        return pl.pallas_call(
            _match_gt_body,
            grid=(rows // _BN,),
            in_specs=[bs_t, bs_z, bs_w, bs_b, bs_g, bs_tab],
            out_specs=out_specs,
            out_shape=out_shape,
        )(t, z, w, b8, gt, tab)
    return pl.pallas_call(
        _match_sm_body,
        grid=(rows // _BN,),
        in_specs=[bs_z, bs_w, bs_b, bs_tab],
        out_specs=out_specs,
        out_shape=out_shape,
    )(z, w, b8, tab)


def _sc_gather3(qn, kn, vn, tail, head):
    mesh = plsc.VectorSubcoreMesh(core_axis_name="c", subcore_axis_name="s")

    @functools.partial(
        pl.kernel,
        mesh=mesh,
        out_type=(
            jax.ShapeDtypeStruct((_E, _D), jnp.float32),
            jax.ShapeDtypeStruct((_E, _D), jnp.float32),
            jax.ShapeDtypeStruct((_E, _D), jnp.float32),
        ),
        scratch_types=[
            pltpu.VMEM((_CH,), jnp.int32),
            pltpu.VMEM((_CH,), jnp.int32),
            pltpu.VMEM((_CH, _D), jnp.float32),
            pltpu.VMEM((_CH, _D), jnp.float32),
            pltpu.VMEM((_CH, _D), jnp.float32),
            pltpu.SemaphoreType.DMA,
            pltpu.SemaphoreType.DMA,
            pltpu.SemaphoreType.DMA,
        ],
    )
    def gath(qn_h, kn_h, vn_h, tail_h, head_h, q_h, k_h, v_h,
             tb, hb, qb, kb, vb, sq, sk, sv):
        wid = lax.axis_index("s") * _NC + lax.axis_index("c")
        count = _NCHT // _NW + jnp.where(wid < _NCHT % _NW, 1, 0)

        def body(j, carry):
            off = (wid + j * _NW) * _CH
            pltpu.sync_copy(tail_h.at[pl.ds(off, _CH)], tb)
            pltpu.sync_copy(head_h.at[pl.ds(off, _CH)], hb)
            cq = pltpu.async_copy(qn_h.at[tb], qb, sq)
            ck = pltpu.async_copy(kn_h.at[hb], kb, sk)
            cv = pltpu.async_copy(vn_h.at[hb], vb, sv)
            cq.wait()
            ck.wait()
            cv.wait()
            pltpu.sync_copy(qb, q_h.at[pl.ds(off, _CH)])
            pltpu.sync_copy(kb, k_h.at[pl.ds(off, _CH)])
            pltpu.sync_copy(vb, v_h.at[pl.ds(off, _CH)])
            return carry

        lax.fori_loop(0, count, body, 0)

    return gath(qn, kn, vn, tail, head)


_PCH = 256
_NPCH = _E // _PCH


def _sc_pack(num, exp, tail=None):
    mesh = plsc.VectorSubcoreMesh(core_axis_name="c", subcore_axis_name="s")

    @functools.partial(
        pl.kernel,
        mesh=mesh,
        out_type=(jax.ShapeDtypeStruct((_E // 8, 128), jnp.float32),) * _NGRP,
        scratch_types=[
            pltpu.VMEM((_PCH, _D), jnp.float32),
            pltpu.VMEM((_PCH, _D), jnp.float32),
        ] + [pltpu.VMEM((_PCH // 8, 128), jnp.float32)] * _NGRP,
    )
    def pack(num_h, exp_h, *rest):
        out_hs = rest[:_NGRP]
        nb, eb = rest[_NGRP], rest[_NGRP + 1]
        pbs = rest[_NGRP + 2:]
        wid = lax.axis_index("s") * _NC + lax.axis_index("c")
        count = _NPCH // _NW + jnp.where(wid < _NPCH % _NW, 1, 0)

        def chunk(j, carry):
            cidx = wid + j * _NW
            noff = pl.multiple_of(cidx * _PCH, 8)
            foff = pl.multiple_of(cidx * (_PCH // 8), 8)
            pltpu.sync_copy(num_h.at[pl.ds(noff, _PCH)], nb)
            pltpu.sync_copy(exp_h.at[pl.ds(noff, _PCH)], eb)

            def edge(e, c2):
                for g in range(_NGRP):
                    src = nb if g < _NGRP - 1 else eb
                    soff = (g * _GW) if g < _NGRP - 1 else 0
                    x = src[e, pl.ds(soff, _GW)]
                    pbs[g][e >> 3, pl.ds((e & 7) * _GW, _GW)] = x
                return c2

            lax.fori_loop(0, _PCH, edge, 0)
            for g in range(_NGRP):
                pltpu.sync_copy(pbs[g], out_hs[g].at[pl.ds(foff, _PCH // 8)])
            return carry

        lax.fori_loop(0, count, chunk, 0)

    return pack(num, exp)


def _sc_scatter_tiles(groups, tail):
    mesh = plsc.VectorSubcoreMesh(core_axis_name="c", subcore_axis_name="s")

    @functools.partial(
        pl.kernel,
        mesh=mesh,
        out_type=jax.ShapeDtypeStruct(
            (_NHALF * _NGRP, _NW, _HNP * _GW // 128, 128), jnp.float32),
        scratch_types=[
            pltpu.VMEM((_SCH,), jnp.int32),
            pltpu.VMEM((_SCH // 8, 128), jnp.float32),
            pltpu.VMEM((_HNP * _GW // 128, 128), jnp.float32),
        ],
    )
    def scat(g0, g1, g2, g3, g4, g5, g6, g7, g8, tail_h, out_h, tb, vb, tab):
        wid = lax.axis_index("s") * _NC + lax.axis_index("c")
        count = _NSCH // _NW + jnp.where(wid < _NSCH % _NW, 1, 0)
        g_hs = (g0, g1, g2, g3, g4, g5, g6, g7, g8)

        for half in range(_NHALF):
            lo = half * _HN
            for g in range(_NGRP):
                p = half * _NGRP + g

                def zrow(r, carry):
                    for c in range(8):
                        tab[r, pl.ds(c * 16, 16)] = jnp.zeros(
                            (16,), jnp.float32)
                    return carry

                lax.fori_loop(0, _HNP * _GW // 128, zrow, 0)

                def chunk(j, carry, g_h=g_hs[g], lo=lo):
                    cidx = wid + j * _NW
                    toff = pl.multiple_of(cidx * _SCH, 8)
                    goff = pl.multiple_of(cidx * (_SCH // 8), 8)
                    pltpu.sync_copy(tail_h.at[pl.ds(toff, _SCH)], tb)
                    pltpu.sync_copy(g_h.at[pl.ds(goff, _SCH // 8)], vb)

                    def grp(gi, c2, lo=lo):
                        tv = tb[pl.ds(gi * 16, 16)]
                        for e in range(16):
                            t_e = tv[e]
                            row = jnp.where(
                                (t_e >= lo) & (t_e < lo + _HN), t_e - lo, _TR)
                            x = vb[gi * 2 + e // 8, pl.ds((e % 8) * 16, 16)]
                            cur = tab[row >> 3, pl.ds((row & 7) * 16, 16)]
                            tab[row >> 3, pl.ds((row & 7) * 16, 16)] = cur + x
                        return c2

                    lax.fori_loop(0, _SCH // 16, grp, 0)
                    return carry

                lax.fori_loop(0, count, chunk, 0)
                pltpu.sync_copy(tab, out_h.at[p, wid, :, :])

    return scat(*groups, tail)


def kernel(init_node_emb, init_edge_emb, head_ind, tail_ind, is_training,
           gt_node_dists, gt_edge_dists, Wq, Wk, Wv, We, Wo, Weo,
           W_ecls, b_ecls, W_ncls, b_ncls, edge_table, node_table):
    tail = tail_ind.astype(jnp.int32)
    head = head_ind.astype(jnp.int32)
    t = jnp.asarray(is_training, jnp.int32).reshape(1, 1)
    be8 = jnp.broadcast_to(b_ecls.reshape(1, _NEC), (8, _NEC))
    bn8 = jnp.broadcast_to(b_ncls.reshape(1, _NNC), (8, _NNC))

    def layer(zn, ze, wq, wk, wv, we, wo, weo):
        qn, kn, vn = _tc_qkv(zn, wq, wk, wv)
        q, k, v = _sc_gather3(qn, kn, vn, tail, head)
        ne, num, exp = _tc_edge(ze, q, k, v, we, weo)
        groups = _sc_pack(num, exp)
        parts = _sc_scatter_tiles(groups, tail)
        psum = _tc_psum(parts).reshape(_NHALF * _NGRP, _HNP, _GW)
        nn = _tc_nodeout(psum, zn, wo)
        return nn, ne

    zn, ze = init_node_emb, init_edge_emb
    for l in range(_L):
        zn, ze = layer(zn, ze, Wq[l], Wk[l], Wv[l], We[l], Wo[l], Weo[l])
    a_e1, d_e1 = _tc_match(ze, W_ecls, be8, gt_edge_dists, edge_table, t, True)
    a_n1, d_n1 = _tc_match(zn, W_ncls, bn8, gt_node_dists, node_table, t, True)
    zn, ze = d_n1, d_e1
    for l in range(_L):
        zn, ze = layer(zn, ze, Wq[l], Wk[l], Wv[l], We[l], Wo[l], Weo[l])
    a_e2, d_e2 = _tc_match(ze, W_ecls, be8, None, edge_table, None, False)
    a_n2, d_n2 = _tc_match(zn, W_ncls, bn8, None, node_table, None, False)
    return (a_e1, a_e2, d_e2, a_n1, a_n2, d_n2)
